# Initial kernel scaffold; baseline (speedup 1.0000x reference)
import jax
import jax.numpy as jnp
from jax.experimental import pallas as pl


def kernel(atom_types, atom_charges, bond_orders, coords, edge_index, a_emb, c_emb, e_emb, ns_w, ns_b, wh, ws, bs, wv, wg, bg, al_w1, al_b1, al_w2, al_b2):
    raise NotImplementedError("write your pallas kernel here")


# speedup vs baseline: 16.1859x; 16.1859x over previous
import functools

import jax
import jax.numpy as jnp
from jax import lax
from jax.experimental import pallas as pl
from jax.experimental.pallas import tpu as pltpu
from jax.experimental.pallas import tpu_sc as plsc

N = 10000
E = 320000
A_DIM = 16
C_DIM = 8
E_DIM = 8
S = 128
VEC = 4
L = 3
LAT = 8
RBF = 10
DMAX = 32.0
N_ATYPE = 10
N_CHG = 6
N_BOND = 5

TG = 256
TM = 128
NW = 32
NSUB = 16
CHUNK = 80
EPT = E // NW
NCH_E = EPT // CHUNK
CPT = (2 * E) // NW
NCH_C = CPT // CHUNK
NPS = 640
NPS_L = N - (NSUB - 1) * NPS

BN = 2000
GN = N // BN
BE = 2000
GE = E // BE

_MESH = dict(core_axis_name="c", subcore_axis_name="s")


def _sc_gather(table, idx3, d, nch):
    rows_total = NW * nch * CHUNK
    mesh = plsc.VectorSubcoreMesh(**_MESH)

    @functools.partial(
        pl.kernel,
        mesh=mesh,
        out_type=jax.ShapeDtypeStruct((rows_total, d), jnp.float32),
        scratch_types=[
            pltpu.VMEM((nch, CHUNK), jnp.int32),
            pltpu.VMEM((CHUNK, d), jnp.float32),
            pltpu.VMEM((CHUNK, d), jnp.float32),
            pltpu.SemaphoreType.DMA,
            pltpu.SemaphoreType.DMA,
        ],
    )
    def k(table_hbm, idx_hbm, out_hbm, idx_v, buf_a, buf_b, sem_a, sem_b):
        wid = lax.axis_index("s") * 2 + lax.axis_index("c")
        base = wid * (nch * CHUNK)
        pltpu.sync_copy(idx_hbm.at[wid], idx_v)

        @pl.loop(0, nch)
        def _(j):
            pltpu.async_copy(table_hbm.at[idx_v.at[j]], buf_a, sem_a).wait()
            pltpu.sync_copy(buf_a, out_hbm.at[pl.ds(base + j * CHUNK, CHUNK)])

    return k(table, idx3)


def _sc_scatter_add(rows, idx, zeros_nd):
    mesh = plsc.VectorSubcoreMesh(**_MESH)

    @functools.partial(
        pl.kernel,
        mesh=mesh,
        out_type=jax.ShapeDtypeStruct((2, N, TM), jnp.float32),
        scratch_types=[
            pltpu.VMEM((CHUNK,), jnp.int32),
            pltpu.VMEM((CHUNK, TM), jnp.float32),
            pltpu.VMEM_SHARED((N, TM), jnp.float32),
            pltpu.SemaphoreType.DMA,
        ],
    )
    def k(rows_hbm, idx_hbm, zeros_hbm, out_hbm, idx_v, buf, acc, sem):
        cid = lax.axis_index("c")
        sid = lax.axis_index("s")
        wid = sid * 2 + cid
        base = wid * (NCH_E * CHUNK)
        @pl.when(sid < NSUB - 1)
        def _():
            pltpu.sync_copy(zeros_hbm.at[pl.ds(sid * NPS, NPS)],
                            acc.at[pl.ds(sid * NPS, NPS)])

        @pl.when(sid == NSUB - 1)
        def _():
            pltpu.sync_copy(zeros_hbm.at[pl.ds((NSUB - 1) * NPS, NPS_L)],
                            acc.at[pl.ds((NSUB - 1) * NPS, NPS_L)])

        plsc.subcore_barrier()

        @pl.loop(0, NCH_E)
        def _(j):
            pltpu.sync_copy(idx_hbm.at[pl.ds(base + j * CHUNK, CHUNK)], idx_v)
            pltpu.sync_copy(rows_hbm.at[pl.ds(base + j * CHUNK, CHUNK)], buf)
            pltpu.sync_copy(buf, acc.at[idx_v], add=True)

        plsc.subcore_barrier()

        @pl.when(sid < NSUB - 1)
        def _():
            pltpu.sync_copy(acc.at[pl.ds(sid * NPS, NPS)],
                            out_hbm.at[cid, pl.ds(sid * NPS, NPS)])

        @pl.when(sid == NSUB - 1)
        def _():
            pltpu.sync_copy(acc.at[pl.ds((NSUB - 1) * NPS, NPS_L)],
                            out_hbm.at[cid, pl.ds((NSUB - 1) * NPS, NPS_L)])

    return k(rows, idx, zeros_nd)


def _embed_body(at_ref, ac_ref, a1_ref, c1_ref, nsb_ref, wss_ref, s_ref, t_ref):
    oh_a = (at_ref[...] == lax.broadcasted_iota(jnp.int32, (1, N_ATYPE), 1)
            ).astype(jnp.float32)
    oh_c = (ac_ref[...] == lax.broadcasted_iota(jnp.int32, (1, N_CHG), 1)
            ).astype(jnp.float32)
    s0 = jnp.maximum(oh_a @ a1_ref[...] + oh_c @ c1_ref[...] + nsb_ref[...], 0.0)
    s_ref[...] = s0
    t = s0 @ wss_ref[...]
    t_ref[...] = jnp.concatenate(
        [t, jnp.zeros((t.shape[0], TG - S), jnp.float32)], axis=1)


def _edge_static_body(cs_ref, cd_ref, bo_ref, es_ref):
    diff = cs_ref[:, :3] - cd_ref[:, :3]
    d = jnp.sqrt(jnp.sum(diff * diff, axis=1, keepdims=True) + 1e-8)
    xd = diff / d
    mu = lax.broadcasted_iota(jnp.int32, (1, RBF), 1).astype(jnp.float32) * (
        DMAX / (RBF - 1))
    sigma = DMAX / RBF
    rbf = jnp.exp(-(((d - mu) / sigma) ** 2))
    oh_b = (bo_ref[...] == lax.broadcasted_iota(jnp.int32, (1, N_BOND), 1)
            ).astype(jnp.float32)
    b = xd.shape[0]
    es_ref[...] = jnp.concatenate(
        [rbf, oh_b, xd, jnp.zeros((b, TM - 18), jnp.float32)], axis=1)


def _edge_body(g_ref, es_ref, wst_ref, bsr_ref, kh_ref, kx_ref, psel_ref,
               wsv_ref, wg_ref, bgr_ref, kv_ref, rg_ref, ms_ref, mv_ref):
    g = g_ref[...]
    es = es_ref[...]
    lin = g[:, :S] + es[:, :15] @ wst_ref[...] + bsr_ref[...]
    vh = g[:, S:S + 12] @ kh_ref[...] + es[:, 15:18] @ kx_ref[...]
    vn = jnp.sqrt((vh * vh) @ psel_ref[...] + 1e-8)
    s_out = jnp.maximum(lin + vn @ wsv_ref[...], 0.0)
    gate = jax.nn.sigmoid(s_out @ wg_ref[...] + bgr_ref[...])
    mv = (vh @ kv_ref[...]) * (gate @ rg_ref[...])
    b = s_out.shape[0]
    ms_ref[...] = s_out
    mv_ref[...] = jnp.concatenate(
        [mv, jnp.ones((b, 1), jnp.float32),
         jnp.zeros((b, TM - 13), jnp.float32)], axis=1)


def _update_body(ps_ref, pv_ref, s_ref, vf_ref, wssn_ref, s_out_ref,
                 vf_out_ref, t_out_ref):
    p = ps_ref[0] + ps_ref[1]
    q = pv_ref[0] + pv_ref[1]
    denom = jnp.maximum(q[:, 12:13], 1.0)
    s_new = s_ref[...] + p / denom
    vf_new = vf_ref[...] + q[:, :12] / denom
    s_out_ref[...] = s_new
    vf_out_ref[...] = vf_new
    t_out_ref[...] = jnp.concatenate(
        [s_new @ wssn_ref[...], vf_new,
         jnp.zeros((s_new.shape[0], TG - S - 12), jnp.float32)], axis=1)


def _final_body(ps_ref, pv_ref, s_ref, w1_ref, b1_ref, w2_ref, b2_ref,
                out_ref):
    p = ps_ref[0] + ps_ref[1]
    q = pv_ref[0] + pv_ref[1]
    denom = jnp.maximum(q[:, 12:13], 1.0)
    s_new = s_ref[...] + p / denom
    h = jnp.maximum(s_new @ w1_ref[...] + b1_ref[...], 0.0)
    out_ref[...] = h @ w2_ref[...] + b2_ref[...]


def _full(shape):
    return pl.BlockSpec(shape, lambda i: tuple(0 for _ in shape))


def kernel(atom_types, atom_charges, bond_orders, coords, edge_index, a_emb,
           c_emb, e_emb, ns_w, ns_b, wh, ws, bs, wv, wg, bg, al_w1, al_b1,
           al_w2, al_b2):
    f32 = jnp.float32
    src = edge_index[0].astype(jnp.int32)
    dst = edge_index[1].astype(jnp.int32)
    at2 = atom_types.astype(jnp.int32).reshape(N, 1)
    ac2 = atom_charges.astype(jnp.int32).reshape(N, 1)
    bo2 = bond_orders.astype(jnp.int32).reshape(E, 1)
    coords_p = jnp.pad(coords.astype(f32), ((0, 0), (0, TM - 3)))

    a1 = a_emb.astype(f32) @ ns_w[:A_DIM]
    c1 = c_emb.astype(f32) @ ns_w[A_DIM:]
    eye3 = jnp.eye(3, dtype=f32)
    wss = [ws[l][:S] for l in range(L)]
    wst = [jnp.concatenate([ws[l][136:146], e_emb @ ws[l][S:136]], axis=0)
           for l in range(L)]
    wsv = [ws[l][146:151] for l in range(L)]
    kh = [jnp.kron(wh[l][:VEC, :], eye3) for l in range(L)]
    kx = [jnp.kron(wh[l][VEC:, :], eye3) for l in range(L)]
    kv = [jnp.kron(wv[l], eye3) for l in range(L)]
    psel = jnp.kron(jnp.eye(VEC + 1, dtype=f32), jnp.ones((3, 1), f32))
    rg = jnp.kron(jnp.eye(VEC, dtype=f32), jnp.ones((1, 3), f32))

    src3 = src.reshape(NW, NCH_E, CHUNK)
    cd3 = jnp.concatenate([src, dst]).reshape(NW, NCH_C, CHUNK)
    zeros_acc = jnp.zeros((N, TM), f32)

    s, t = pl.pallas_call(
        _embed_body,
        grid=(GN,),
        in_specs=[
            pl.BlockSpec((BN, 1), lambda i: (i, 0)),
            pl.BlockSpec((BN, 1), lambda i: (i, 0)),
            _full((N_ATYPE, S)),
            _full((N_CHG, S)),
            _full((1, S)),
            _full((S, S)),
        ],
        out_specs=[pl.BlockSpec((BN, S), lambda i: (i, 0)),
                   pl.BlockSpec((BN, TG), lambda i: (i, 0))],
        out_shape=[jax.ShapeDtypeStruct((N, S), f32),
                   jax.ShapeDtypeStruct((N, TG), f32)],
    )(at2, ac2, a1, c1, ns_b.reshape(1, S), wss[0])

    gc = _sc_gather(coords_p, cd3, TM, NCH_C)
    t, _gc_done = lax.optimization_barrier((t, gc))

    es = pl.pallas_call(
        _edge_static_body,
        grid=(GE,),
        in_specs=[
            pl.BlockSpec((BE, TM), lambda i: (i, 0)),
            pl.BlockSpec((BE, TM), lambda i: (i + GE, 0)),
            pl.BlockSpec((BE, 1), lambda i: (i, 0)),
        ],
        out_specs=pl.BlockSpec((BE, TM), lambda i: (i, 0)),
        out_shape=jax.ShapeDtypeStruct((E, TM), f32),
    )(gc, gc, bo2)

    vf = jnp.zeros((N, 12), f32)
    out = None
    for l in range(L):
        g = _sc_gather(t, src3, TG, NCH_E)
        ms, mv = pl.pallas_call(
            _edge_body,
            grid=(GE,),
            in_specs=[
                pl.BlockSpec((BE, TG), lambda i: (i, 0)),
                pl.BlockSpec((BE, TM), lambda i: (i, 0)),
                _full((15, S)),
                _full((1, S)),
                _full((12, 15)),
                _full((3, 15)),
                _full((15, VEC + 1)),
                _full((VEC + 1, S)),
                _full((S, VEC)),
                _full((1, VEC)),
                _full((15, 12)),
                _full((VEC, 12)),
            ],
            out_specs=[pl.BlockSpec((BE, TM), lambda i: (i, 0)),
                       pl.BlockSpec((BE, TM), lambda i: (i, 0))],
            out_shape=[jax.ShapeDtypeStruct((E, TM), f32),
                       jax.ShapeDtypeStruct((E, TM), f32)],
        )(g, es, wst[l], bs[l].reshape(1, S), kh[l], kx[l], psel, wsv[l],
          wg[l], bg[l].reshape(1, VEC), kv[l], rg)

        part_s = _sc_scatter_add(ms, dst, zeros_acc)
        mv, _ps_done = lax.optimization_barrier((mv, part_s))
        part_v = _sc_scatter_add(mv, dst, zeros_acc)

        if l < L - 1:
            s, vf, t = pl.pallas_call(
                _update_body,
                grid=(GN,),
                in_specs=[
                    pl.BlockSpec((2, BN, TM), lambda i: (0, i, 0)),
                    pl.BlockSpec((2, BN, TM), lambda i: (0, i, 0)),
                    pl.BlockSpec((BN, S), lambda i: (i, 0)),
                    pl.BlockSpec((BN, 12), lambda i: (i, 0)),
                    _full((S, S)),
                ],
                out_specs=[pl.BlockSpec((BN, S), lambda i: (i, 0)),
                           pl.BlockSpec((BN, 12), lambda i: (i, 0)),
                           pl.BlockSpec((BN, TG), lambda i: (i, 0))],
                out_shape=[jax.ShapeDtypeStruct((N, S), f32),
                           jax.ShapeDtypeStruct((N, 12), f32),
                           jax.ShapeDtypeStruct((N, TG), f32)],
            )(part_s, part_v, s, vf, wss[l + 1])
        else:
            out = pl.pallas_call(
                _final_body,
                grid=(GN,),
                in_specs=[
                    pl.BlockSpec((2, BN, TM), lambda i: (0, i, 0)),
                    pl.BlockSpec((2, BN, TM), lambda i: (0, i, 0)),
                    pl.BlockSpec((BN, S), lambda i: (i, 0)),
                    _full((S, 2 * LAT)),
                    _full((1, 2 * LAT)),
                    _full((2 * LAT, LAT)),
                    _full((1, LAT)),
                ],
                out_specs=pl.BlockSpec((BN, LAT), lambda i: (i, 0)),
                out_shape=jax.ShapeDtypeStruct((N, LAT), f32),
            )(part_s, part_v, s, al_w1, al_b1.reshape(1, 2 * LAT), al_w2,
              al_b2.reshape(1, LAT))
    return out


# speedup vs baseline: 16.2039x; 1.0011x over previous
import functools

import jax
import jax.numpy as jnp
from jax import lax
from jax.experimental import pallas as pl
from jax.experimental.pallas import tpu as pltpu
from jax.experimental.pallas import tpu_sc as plsc

N = 10000
E = 320000
A_DIM = 16
C_DIM = 8
E_DIM = 8
S = 128
VEC = 4
L = 3
LAT = 8
RBF = 10
DMAX = 32.0
N_ATYPE = 10
N_CHG = 6
N_BOND = 5

TG = 256
TM = 128
TV = 16
EW = 32
NW = 32
NSUB = 16
CHUNK = 80
EPT = E // NW
NCH_E = EPT // CHUNK
CPT = (2 * E) // NW
NCH_C = CPT // CHUNK
NPS = 640
NPS_L = N - (NSUB - 1) * NPS

BN = 2000
GN = N // BN
BE = 2000
GE = E // BE

_MESH = dict(core_axis_name="c", subcore_axis_name="s")


def _sc_gather(table, idx3, d, nch, d_out=None):
    d_out = d if d_out is None else d_out
    rows_total = NW * nch * CHUNK
    mesh = plsc.VectorSubcoreMesh(**_MESH)

    @functools.partial(
        pl.kernel,
        mesh=mesh,
        out_type=jax.ShapeDtypeStruct((rows_total, d_out), jnp.float32),
        scratch_types=[
            pltpu.VMEM((nch, CHUNK), jnp.int32),
            pltpu.VMEM((CHUNK, d), jnp.float32),
            pltpu.VMEM((CHUNK, d), jnp.float32),
            pltpu.SemaphoreType.DMA,
            pltpu.SemaphoreType.DMA,
        ],
    )
    def k(table_hbm, idx_hbm, out_hbm, idx_v, buf_a, buf_b, sem_a, sem_b):
        wid = lax.axis_index("s") * 2 + lax.axis_index("c")
        base = wid * (nch * CHUNK)
        pltpu.sync_copy(idx_hbm.at[wid], idx_v)

        @pl.loop(0, nch)
        def _(j):
            pltpu.async_copy(table_hbm.at[idx_v.at[j]], buf_a, sem_a).wait()
            pltpu.sync_copy(buf_a, out_hbm.at[pl.ds(base + j * CHUNK, CHUNK)])

    return k(table, idx3)


def _sc_scatter_add(rows, idx, zeros_nd, d):
    mesh = plsc.VectorSubcoreMesh(**_MESH)
    kwargs = {}
    if d < TM:
        kwargs["compiler_params"] = pltpu.CompilerParams(
            use_tc_tiling_on_sc=False)

    @functools.partial(
        pl.kernel,
        mesh=mesh,
        out_type=jax.ShapeDtypeStruct((2, N, d), jnp.float32),
        scratch_types=[
            pltpu.VMEM((CHUNK,), jnp.int32),
            pltpu.VMEM((CHUNK, d), jnp.float32),
            pltpu.VMEM_SHARED((N, d), jnp.float32),
            pltpu.SemaphoreType.DMA,
        ],
        **kwargs,
    )
    def k(rows_hbm, idx_hbm, zeros_hbm, out_hbm, idx_v, buf, acc, sem):
        cid = lax.axis_index("c")
        sid = lax.axis_index("s")
        wid = sid * 2 + cid
        base = wid * (NCH_E * CHUNK)
        @pl.when(sid < NSUB - 1)
        def _():
            pltpu.sync_copy(zeros_hbm.at[pl.ds(sid * NPS, NPS)],
                            acc.at[pl.ds(sid * NPS, NPS)])

        @pl.when(sid == NSUB - 1)
        def _():
            pltpu.sync_copy(zeros_hbm.at[pl.ds((NSUB - 1) * NPS, NPS_L)],
                            acc.at[pl.ds((NSUB - 1) * NPS, NPS_L)])

        plsc.subcore_barrier()

        @pl.loop(0, NCH_E)
        def _(j):
            pltpu.sync_copy(idx_hbm.at[pl.ds(base + j * CHUNK, CHUNK)], idx_v)
            pltpu.sync_copy(rows_hbm.at[pl.ds(base + j * CHUNK, CHUNK)], buf)
            pltpu.sync_copy(buf, acc.at[idx_v], add=True)

        plsc.subcore_barrier()

        @pl.when(sid < NSUB - 1)
        def _():
            pltpu.sync_copy(acc.at[pl.ds(sid * NPS, NPS)],
                            out_hbm.at[cid, pl.ds(sid * NPS, NPS)])

        @pl.when(sid == NSUB - 1)
        def _():
            pltpu.sync_copy(acc.at[pl.ds((NSUB - 1) * NPS, NPS_L)],
                            out_hbm.at[cid, pl.ds((NSUB - 1) * NPS, NPS_L)])

    return k(rows, idx, zeros_nd)


def _embed_body(at_ref, ac_ref, a1_ref, c1_ref, nsb_ref, wss_ref, s_ref, t_ref):
    oh_a = (at_ref[...] == lax.broadcasted_iota(jnp.int32, (1, N_ATYPE), 1)
            ).astype(jnp.float32)
    oh_c = (ac_ref[...] == lax.broadcasted_iota(jnp.int32, (1, N_CHG), 1)
            ).astype(jnp.float32)
    s0 = jnp.maximum(oh_a @ a1_ref[...] + oh_c @ c1_ref[...] + nsb_ref[...], 0.0)
    s_ref[...] = s0
    t = s0 @ wss_ref[...]
    t_ref[...] = jnp.concatenate(
        [t, jnp.zeros((t.shape[0], TG - S), jnp.float32)], axis=1)


def _edge_static_body(cs_ref, cd_ref, bo_ref, es_ref):
    diff = cs_ref[:, :3] - cd_ref[:, :3]
    d = jnp.sqrt(jnp.sum(diff * diff, axis=1, keepdims=True) + 1e-8)
    xd = diff / d
    mu = lax.broadcasted_iota(jnp.int32, (1, RBF), 1).astype(jnp.float32) * (
        DMAX / (RBF - 1))
    sigma = DMAX / RBF
    rbf = jnp.exp(-(((d - mu) / sigma) ** 2))
    oh_b = (bo_ref[...] == lax.broadcasted_iota(jnp.int32, (1, N_BOND), 1)
            ).astype(jnp.float32)
    b = xd.shape[0]
    es_ref[...] = jnp.concatenate(
        [rbf, oh_b, xd, jnp.zeros((b, EW - 18), jnp.float32)], axis=1)


def _edge_body(g_ref, es_ref, wst_ref, bsr_ref, kh_ref, kx_ref, psel_ref,
               wsv_ref, wg_ref, bgr_ref, kv_ref, rg_ref, ms_ref, mv_ref):
    g = g_ref[...]
    es = es_ref[...]
    lin = g[:, :S] + es[:, :15] @ wst_ref[...] + bsr_ref[...]
    vh = g[:, S:S + 12] @ kh_ref[...] + es[:, 15:18] @ kx_ref[...]
    vn = jnp.sqrt((vh * vh) @ psel_ref[...] + 1e-8)
    s_out = jnp.maximum(lin + vn @ wsv_ref[...], 0.0)
    gate = jax.nn.sigmoid(s_out @ wg_ref[...] + bgr_ref[...])
    mv = (vh @ kv_ref[...]) * (gate @ rg_ref[...])
    b = s_out.shape[0]
    ms_ref[...] = s_out
    mv_ref[...] = jnp.concatenate(
        [mv, jnp.ones((b, 1), jnp.float32),
         jnp.zeros((b, TV - 13), jnp.float32)], axis=1)


def _update_body(ps_ref, pv_ref, s_ref, vf_ref, wssn_ref, s_out_ref,
                 vf_out_ref, t_out_ref):
    p = ps_ref[0] + ps_ref[1]
    q = pv_ref[0] + pv_ref[1]
    denom = jnp.maximum(q[:, 12:13], 1.0)
    s_new = s_ref[...] + p / denom
    vf_new = vf_ref[...] + q[:, :12] / denom
    s_out_ref[...] = s_new
    vf_out_ref[...] = vf_new
    t_out_ref[...] = jnp.concatenate(
        [s_new @ wssn_ref[...], vf_new,
         jnp.zeros((s_new.shape[0], TG - S - 12), jnp.float32)], axis=1)


def _final_body(ps_ref, pv_ref, s_ref, w1_ref, b1_ref, w2_ref, b2_ref,
                out_ref):
    p = ps_ref[0] + ps_ref[1]
    q = pv_ref[0] + pv_ref[1]
    denom = jnp.maximum(q[:, 12:13], 1.0)
    s_new = s_ref[...] + p / denom
    h = jnp.maximum(s_new @ w1_ref[...] + b1_ref[...], 0.0)
    out_ref[...] = h @ w2_ref[...] + b2_ref[...]


def _full(shape):
    return pl.BlockSpec(shape, lambda i: tuple(0 for _ in shape))


def kernel(atom_types, atom_charges, bond_orders, coords, edge_index, a_emb,
           c_emb, e_emb, ns_w, ns_b, wh, ws, bs, wv, wg, bg, al_w1, al_b1,
           al_w2, al_b2):
    f32 = jnp.float32
    src = edge_index[0].astype(jnp.int32)
    dst = edge_index[1].astype(jnp.int32)
    at2 = atom_types.astype(jnp.int32).reshape(N, 1)
    ac2 = atom_charges.astype(jnp.int32).reshape(N, 1)
    bo2 = bond_orders.astype(jnp.int32).reshape(E, 1)
    coords_p = jnp.pad(coords.astype(f32), ((0, 0), (0, TM - 3)))

    a1 = a_emb.astype(f32) @ ns_w[:A_DIM]
    c1 = c_emb.astype(f32) @ ns_w[A_DIM:]
    eye3 = jnp.eye(3, dtype=f32)
    wss = [ws[l][:S] for l in range(L)]
    wst = [jnp.concatenate([ws[l][136:146], e_emb @ ws[l][S:136]], axis=0)
           for l in range(L)]
    wsv = [ws[l][146:151] for l in range(L)]
    kh = [jnp.kron(wh[l][:VEC, :], eye3) for l in range(L)]
    kx = [jnp.kron(wh[l][VEC:, :], eye3) for l in range(L)]
    kv = [jnp.kron(wv[l], eye3) for l in range(L)]
    psel = jnp.kron(jnp.eye(VEC + 1, dtype=f32), jnp.ones((3, 1), f32))
    rg = jnp.kron(jnp.eye(VEC, dtype=f32), jnp.ones((1, 3), f32))

    src3 = src.reshape(NW, NCH_E, CHUNK)
    cd3 = jnp.concatenate([src, dst]).reshape(NW, NCH_C, CHUNK)
    zeros_s = jnp.zeros((N, TM), f32)
    zeros_v = jnp.zeros((N, TV), f32)

    s, t = pl.pallas_call(
        _embed_body,
        grid=(GN,),
        in_specs=[
            pl.BlockSpec((BN, 1), lambda i: (i, 0)),
            pl.BlockSpec((BN, 1), lambda i: (i, 0)),
            _full((N_ATYPE, S)),
            _full((N_CHG, S)),
            _full((1, S)),
            _full((S, S)),
        ],
        out_specs=[pl.BlockSpec((BN, S), lambda i: (i, 0)),
                   pl.BlockSpec((BN, TG), lambda i: (i, 0))],
        out_shape=[jax.ShapeDtypeStruct((N, S), f32),
                   jax.ShapeDtypeStruct((N, TG), f32)],
    )(at2, ac2, a1, c1, ns_b.reshape(1, S), wss[0])

    gc = _sc_gather(coords_p, cd3, TM, NCH_C)
    t, _gc_done = lax.optimization_barrier((t, gc))

    es = pl.pallas_call(
        _edge_static_body,
        grid=(GE,),
        in_specs=[
            pl.BlockSpec((BE, TM), lambda i: (i, 0)),
            pl.BlockSpec((BE, TM), lambda i: (i + GE, 0)),
            pl.BlockSpec((BE, 1), lambda i: (i, 0)),
        ],
        out_specs=pl.BlockSpec((BE, EW), lambda i: (i, 0)),
        out_shape=jax.ShapeDtypeStruct((E, EW), f32),
    )(gc, gc, bo2)

    vf = jnp.zeros((N, 12), f32)
    out = None
    for l in range(L):
        g = _sc_gather(t, src3, TG, NCH_E)
        ms, mv = pl.pallas_call(
            _edge_body,
            grid=(GE,),
            in_specs=[
                pl.BlockSpec((BE, TG), lambda i: (i, 0)),
                pl.BlockSpec((BE, EW), lambda i: (i, 0)),
                _full((15, S)),
                _full((1, S)),
                _full((12, 15)),
                _full((3, 15)),
                _full((15, VEC + 1)),
                _full((VEC + 1, S)),
                _full((S, VEC)),
                _full((1, VEC)),
                _full((15, 12)),
                _full((VEC, 12)),
            ],
            out_specs=[pl.BlockSpec((BE, TM), lambda i: (i, 0)),
                       pl.BlockSpec((BE, TV), lambda i: (i, 0))],
            out_shape=[jax.ShapeDtypeStruct((E, TM), f32),
                       jax.ShapeDtypeStruct((E, TV), f32)],
        )(g, es, wst[l], bs[l].reshape(1, S), kh[l], kx[l], psel, wsv[l],
          wg[l], bg[l].reshape(1, VEC), kv[l], rg)

        part_s = _sc_scatter_add(ms, dst, zeros_s, TM)
        mv, _ps_done = lax.optimization_barrier((mv, part_s))
        part_v = _sc_scatter_add(mv, dst, zeros_v, TV)

        if l < L - 1:
            s, vf, t = pl.pallas_call(
                _update_body,
                grid=(GN,),
                in_specs=[
                    pl.BlockSpec((2, BN, TM), lambda i: (0, i, 0)),
                    pl.BlockSpec((2, BN, TV), lambda i: (0, i, 0)),
                    pl.BlockSpec((BN, S), lambda i: (i, 0)),
                    pl.BlockSpec((BN, 12), lambda i: (i, 0)),
                    _full((S, S)),
                ],
                out_specs=[pl.BlockSpec((BN, S), lambda i: (i, 0)),
                           pl.BlockSpec((BN, 12), lambda i: (i, 0)),
                           pl.BlockSpec((BN, TG), lambda i: (i, 0))],
                out_shape=[jax.ShapeDtypeStruct((N, S), f32),
                           jax.ShapeDtypeStruct((N, 12), f32),
                           jax.ShapeDtypeStruct((N, TG), f32)],
            )(part_s, part_v, s, vf, wss[l + 1])
        else:
            out = pl.pallas_call(
                _final_body,
                grid=(GN,),
                in_specs=[
                    pl.BlockSpec((2, BN, TM), lambda i: (0, i, 0)),
                    pl.BlockSpec((2, BN, TV), lambda i: (0, i, 0)),
                    pl.BlockSpec((BN, S), lambda i: (i, 0)),
                    _full((S, 2 * LAT)),
                    _full((1, 2 * LAT)),
                    _full((2 * LAT, LAT)),
                    _full((1, LAT)),
                ],
                out_specs=pl.BlockSpec((BN, LAT), lambda i: (i, 0)),
                out_shape=jax.ShapeDtypeStruct((N, LAT), f32),
            )(part_s, part_v, s, al_w1, al_b1.reshape(1, 2 * LAT), al_w2,
              al_b2.reshape(1, LAT))
    return out


# speedup vs baseline: 19.3390x; 1.1935x over previous
import functools

import jax
import jax.numpy as jnp
from jax import lax
from jax.experimental import pallas as pl
from jax.experimental.pallas import tpu as pltpu
from jax.experimental.pallas import tpu_sc as plsc

N = 10000
E = 320000
A_DIM = 16
C_DIM = 8
E_DIM = 8
S = 128
VEC = 4
L = 3
LAT = 8
RBF = 10
DMAX = 32.0
N_ATYPE = 10
N_CHG = 6
N_BOND = 5

TG = 256
TM = 128
TV = 16
EW = 32
NW = 32
NSUB = 16
CHUNK = 80
EPT = E // NW
NCH_E = EPT // CHUNK
CPT = (2 * E) // NW
NCH_C = CPT // CHUNK
NPS = 640
NPS_L = N - (NSUB - 1) * NPS

BN = 2000
GN = N // BN
BE = 2000
GE = E // BE

_MESH = dict(core_axis_name="c", subcore_axis_name="s")


def _sc_gather(table, idx3, d, nch, d_out=None):
    d_out = d if d_out is None else d_out
    rows_total = NW * nch * CHUNK
    mesh = plsc.VectorSubcoreMesh(**_MESH)

    @functools.partial(
        pl.kernel,
        mesh=mesh,
        out_type=jax.ShapeDtypeStruct((rows_total, d_out), jnp.float32),
        scratch_types=[
            pltpu.VMEM((nch, CHUNK), jnp.int32),
            pltpu.VMEM((CHUNK, d), jnp.float32),
            pltpu.VMEM((CHUNK, d), jnp.float32),
            pltpu.SemaphoreType.DMA,
            pltpu.SemaphoreType.DMA,
        ],
    )
    def k(table_hbm, idx_hbm, out_hbm, idx_v, buf_a, buf_b, sem_a, sem_b):
        wid = lax.axis_index("s") * 2 + lax.axis_index("c")
        base = wid * (nch * CHUNK)
        pltpu.sync_copy(idx_hbm.at[wid], idx_v)
        pltpu.async_copy(table_hbm.at[idx_v.at[0]], buf_a, sem_a)

        @pl.loop(0, nch // 2)
        def _(p):
            j = 2 * p
            pltpu.make_async_copy(table_hbm.at[idx_v.at[j]], buf_a, sem_a).wait()
            pltpu.async_copy(table_hbm.at[idx_v.at[j + 1]], buf_b, sem_b)
            pltpu.sync_copy(buf_a, out_hbm.at[pl.ds(base + j * CHUNK, CHUNK)])
            pltpu.make_async_copy(table_hbm.at[idx_v.at[j + 1]], buf_b,
                                  sem_b).wait()

            @pl.when(j + 2 < nch)
            def _():
                pltpu.async_copy(table_hbm.at[idx_v.at[j + 2]], buf_a, sem_a)

            pltpu.sync_copy(buf_b,
                            out_hbm.at[pl.ds(base + (j + 1) * CHUNK, CHUNK)])

        if nch % 2:
            jt = nch - 1
            pltpu.make_async_copy(table_hbm.at[idx_v.at[jt]], buf_a,
                                  sem_a).wait()
            pltpu.sync_copy(buf_a, out_hbm.at[pl.ds(base + jt * CHUNK, CHUNK)])

    return k(table, idx3)


def _sc_scatter_add(rows, idx, zeros_nd, d):
    mesh = plsc.VectorSubcoreMesh(**_MESH)
    kwargs = {}
    if d < TM:
        kwargs["compiler_params"] = pltpu.CompilerParams(
            use_tc_tiling_on_sc=False)

    @functools.partial(
        pl.kernel,
        mesh=mesh,
        out_type=jax.ShapeDtypeStruct((2, N, d), jnp.float32),
        scratch_types=[
            pltpu.VMEM((CHUNK,), jnp.int32),
            pltpu.VMEM((CHUNK,), jnp.int32),
            pltpu.VMEM((CHUNK, d), jnp.float32),
            pltpu.VMEM((CHUNK, d), jnp.float32),
            pltpu.VMEM_SHARED((N, d), jnp.float32),
            pltpu.SemaphoreType.DMA,
            pltpu.SemaphoreType.DMA,
            pltpu.SemaphoreType.DMA,
            pltpu.SemaphoreType.DMA,
        ],
        **kwargs,
    )
    def k(rows_hbm, idx_hbm, zeros_hbm, out_hbm, idx_a, idx_b, buf_a, buf_b,
          acc, sem_ia, sem_ib, sem_a, sem_b):
        cid = lax.axis_index("c")
        sid = lax.axis_index("s")
        wid = sid * 2 + cid
        base = wid * (NCH_E * CHUNK)
        @pl.when(sid < NSUB - 1)
        def _():
            pltpu.sync_copy(zeros_hbm.at[pl.ds(sid * NPS, NPS)],
                            acc.at[pl.ds(sid * NPS, NPS)])

        @pl.when(sid == NSUB - 1)
        def _():
            pltpu.sync_copy(zeros_hbm.at[pl.ds((NSUB - 1) * NPS, NPS_L)],
                            acc.at[pl.ds((NSUB - 1) * NPS, NPS_L)])

        pltpu.async_copy(idx_hbm.at[pl.ds(base, CHUNK)], idx_a, sem_ia)
        pltpu.async_copy(rows_hbm.at[pl.ds(base, CHUNK)], buf_a, sem_a)
        plsc.subcore_barrier()

        @pl.loop(0, NCH_E // 2)
        def _(p):
            j = 2 * p
            o0 = base + j * CHUNK
            o1 = base + (j + 1) * CHUNK
            pltpu.make_async_copy(idx_hbm.at[pl.ds(o0, CHUNK)], idx_a,
                                  sem_ia).wait()
            pltpu.make_async_copy(rows_hbm.at[pl.ds(o0, CHUNK)], buf_a,
                                  sem_a).wait()
            pltpu.async_copy(idx_hbm.at[pl.ds(o1, CHUNK)], idx_b, sem_ib)
            pltpu.async_copy(rows_hbm.at[pl.ds(o1, CHUNK)], buf_b, sem_b)
            pltpu.sync_copy(buf_a, acc.at[idx_a], add=True)
            pltpu.make_async_copy(idx_hbm.at[pl.ds(o1, CHUNK)], idx_b,
                                  sem_ib).wait()
            pltpu.make_async_copy(rows_hbm.at[pl.ds(o1, CHUNK)], buf_b,
                                  sem_b).wait()

            @pl.when(j + 2 < NCH_E)
            def _():
                o2 = base + (j + 2) * CHUNK
                pltpu.async_copy(idx_hbm.at[pl.ds(o2, CHUNK)], idx_a, sem_ia)
                pltpu.async_copy(rows_hbm.at[pl.ds(o2, CHUNK)], buf_a, sem_a)

            pltpu.sync_copy(buf_b, acc.at[idx_b], add=True)

        if NCH_E % 2:
            ot = base + (NCH_E - 1) * CHUNK
            pltpu.make_async_copy(idx_hbm.at[pl.ds(ot, CHUNK)], idx_a,
                                  sem_ia).wait()
            pltpu.make_async_copy(rows_hbm.at[pl.ds(ot, CHUNK)], buf_a,
                                  sem_a).wait()
            pltpu.sync_copy(buf_a, acc.at[idx_a], add=True)

        plsc.subcore_barrier()

        @pl.when(sid < NSUB - 1)
        def _():
            pltpu.sync_copy(acc.at[pl.ds(sid * NPS, NPS)],
                            out_hbm.at[cid, pl.ds(sid * NPS, NPS)])

        @pl.when(sid == NSUB - 1)
        def _():
            pltpu.sync_copy(acc.at[pl.ds((NSUB - 1) * NPS, NPS_L)],
                            out_hbm.at[cid, pl.ds((NSUB - 1) * NPS, NPS_L)])

    return k(rows, idx, zeros_nd)


def _embed_body(at_ref, ac_ref, a1_ref, c1_ref, nsb_ref, wss_ref, s_ref, t_ref):
    oh_a = (at_ref[...] == lax.broadcasted_iota(jnp.int32, (1, N_ATYPE), 1)
            ).astype(jnp.float32)
    oh_c = (ac_ref[...] == lax.broadcasted_iota(jnp.int32, (1, N_CHG), 1)
            ).astype(jnp.float32)
    s0 = jnp.maximum(oh_a @ a1_ref[...] + oh_c @ c1_ref[...] + nsb_ref[...], 0.0)
    s_ref[...] = s0
    t = s0 @ wss_ref[...]
    t_ref[...] = jnp.concatenate(
        [t, jnp.zeros((t.shape[0], TG - S), jnp.float32)], axis=1)


def _edge_static_body(cs_ref, cd_ref, bo_ref, es_ref):
    diff = cs_ref[:, :3] - cd_ref[:, :3]
    d = jnp.sqrt(jnp.sum(diff * diff, axis=1, keepdims=True) + 1e-8)
    xd = diff / d
    mu = lax.broadcasted_iota(jnp.int32, (1, RBF), 1).astype(jnp.float32) * (
        DMAX / (RBF - 1))
    sigma = DMAX / RBF
    rbf = jnp.exp(-(((d - mu) / sigma) ** 2))
    oh_b = (bo_ref[...] == lax.broadcasted_iota(jnp.int32, (1, N_BOND), 1)
            ).astype(jnp.float32)
    b = xd.shape[0]
    es_ref[...] = jnp.concatenate(
        [rbf, oh_b, xd, jnp.zeros((b, EW - 18), jnp.float32)], axis=1)


def _edge_body(g_ref, es_ref, wst_ref, bsr_ref, kh_ref, kx_ref, psel_ref,
               wsv_ref, wg_ref, bgr_ref, kv_ref, rg_ref, ms_ref, mv_ref):
    g = g_ref[...]
    es = es_ref[...]
    lin = g[:, :S] + es[:, :15] @ wst_ref[...] + bsr_ref[...]
    vh = g[:, S:S + 12] @ kh_ref[...] + es[:, 15:18] @ kx_ref[...]
    vn = jnp.sqrt((vh * vh) @ psel_ref[...] + 1e-8)
    s_out = jnp.maximum(lin + vn @ wsv_ref[...], 0.0)
    gate = jax.nn.sigmoid(s_out @ wg_ref[...] + bgr_ref[...])
    mv = (vh @ kv_ref[...]) * (gate @ rg_ref[...])
    b = s_out.shape[0]
    ms_ref[...] = s_out
    mv_ref[...] = jnp.concatenate(
        [mv, jnp.ones((b, 1), jnp.float32),
         jnp.zeros((b, TV - 13), jnp.float32)], axis=1)


def _update_body(ps_ref, pv_ref, s_ref, vf_ref, wssn_ref, s_out_ref,
                 vf_out_ref, t_out_ref):
    p = ps_ref[0] + ps_ref[1]
    q = pv_ref[0] + pv_ref[1]
    denom = jnp.maximum(q[:, 12:13], 1.0)
    s_new = s_ref[...] + p / denom
    vf_new = vf_ref[...] + q[:, :12] / denom
    s_out_ref[...] = s_new
    vf_out_ref[...] = vf_new
    t_out_ref[...] = jnp.concatenate(
        [s_new @ wssn_ref[...], vf_new,
         jnp.zeros((s_new.shape[0], TG - S - 12), jnp.float32)], axis=1)


def _final_body(ps_ref, pv_ref, s_ref, w1_ref, b1_ref, w2_ref, b2_ref,
                out_ref):
    p = ps_ref[0] + ps_ref[1]
    q = pv_ref[0] + pv_ref[1]
    denom = jnp.maximum(q[:, 12:13], 1.0)
    s_new = s_ref[...] + p / denom
    h = jnp.maximum(s_new @ w1_ref[...] + b1_ref[...], 0.0)
    out_ref[...] = h @ w2_ref[...] + b2_ref[...]


def _full(shape):
    return pl.BlockSpec(shape, lambda i: tuple(0 for _ in shape))


def kernel(atom_types, atom_charges, bond_orders, coords, edge_index, a_emb,
           c_emb, e_emb, ns_w, ns_b, wh, ws, bs, wv, wg, bg, al_w1, al_b1,
           al_w2, al_b2):
    f32 = jnp.float32
    src = edge_index[0].astype(jnp.int32)
    dst = edge_index[1].astype(jnp.int32)
    at2 = atom_types.astype(jnp.int32).reshape(N, 1)
    ac2 = atom_charges.astype(jnp.int32).reshape(N, 1)
    bo2 = bond_orders.astype(jnp.int32).reshape(E, 1)
    coords_p = jnp.pad(coords.astype(f32), ((0, 0), (0, TM - 3)))

    a1 = a_emb.astype(f32) @ ns_w[:A_DIM]
    c1 = c_emb.astype(f32) @ ns_w[A_DIM:]
    eye3 = jnp.eye(3, dtype=f32)
    wss = [ws[l][:S] for l in range(L)]
    wst = [jnp.concatenate([ws[l][136:146], e_emb @ ws[l][S:136]], axis=0)
           for l in range(L)]
    wsv = [ws[l][146:151] for l in range(L)]
    kh = [jnp.kron(wh[l][:VEC, :], eye3) for l in range(L)]
    kx = [jnp.kron(wh[l][VEC:, :], eye3) for l in range(L)]
    kv = [jnp.kron(wv[l], eye3) for l in range(L)]
    psel = jnp.kron(jnp.eye(VEC + 1, dtype=f32), jnp.ones((3, 1), f32))
    rg = jnp.kron(jnp.eye(VEC, dtype=f32), jnp.ones((1, 3), f32))

    src3 = src.reshape(NW, NCH_E, CHUNK)
    cd3 = jnp.concatenate([src, dst]).reshape(NW, NCH_C, CHUNK)
    zeros_s = jnp.zeros((N, TM), f32)
    zeros_v = jnp.zeros((N, TV), f32)

    s, t = pl.pallas_call(
        _embed_body,
        grid=(GN,),
        in_specs=[
            pl.BlockSpec((BN, 1), lambda i: (i, 0)),
            pl.BlockSpec((BN, 1), lambda i: (i, 0)),
            _full((N_ATYPE, S)),
            _full((N_CHG, S)),
            _full((1, S)),
            _full((S, S)),
        ],
        out_specs=[pl.BlockSpec((BN, S), lambda i: (i, 0)),
                   pl.BlockSpec((BN, TG), lambda i: (i, 0))],
        out_shape=[jax.ShapeDtypeStruct((N, S), f32),
                   jax.ShapeDtypeStruct((N, TG), f32)],
    )(at2, ac2, a1, c1, ns_b.reshape(1, S), wss[0])

    gc = _sc_gather(coords_p, cd3, TM, NCH_C)
    t, _gc_done = lax.optimization_barrier((t, gc))

    es = pl.pallas_call(
        _edge_static_body,
        grid=(GE,),
        in_specs=[
            pl.BlockSpec((BE, TM), lambda i: (i, 0)),
            pl.BlockSpec((BE, TM), lambda i: (i + GE, 0)),
            pl.BlockSpec((BE, 1), lambda i: (i, 0)),
        ],
        out_specs=pl.BlockSpec((BE, EW), lambda i: (i, 0)),
        out_shape=jax.ShapeDtypeStruct((E, EW), f32),
    )(gc, gc, bo2)

    vf = jnp.zeros((N, 12), f32)
    out = None
    for l in range(L):
        g = _sc_gather(t, src3, TG, NCH_E)
        ms, mv = pl.pallas_call(
            _edge_body,
            grid=(GE,),
            in_specs=[
                pl.BlockSpec((BE, TG), lambda i: (i, 0)),
                pl.BlockSpec((BE, EW), lambda i: (i, 0)),
                _full((15, S)),
                _full((1, S)),
                _full((12, 15)),
                _full((3, 15)),
                _full((15, VEC + 1)),
                _full((VEC + 1, S)),
                _full((S, VEC)),
                _full((1, VEC)),
                _full((15, 12)),
                _full((VEC, 12)),
            ],
            out_specs=[pl.BlockSpec((BE, TM), lambda i: (i, 0)),
                       pl.BlockSpec((BE, TV), lambda i: (i, 0))],
            out_shape=[jax.ShapeDtypeStruct((E, TM), f32),
                       jax.ShapeDtypeStruct((E, TV), f32)],
        )(g, es, wst[l], bs[l].reshape(1, S), kh[l], kx[l], psel, wsv[l],
          wg[l], bg[l].reshape(1, VEC), kv[l], rg)

        part_s = _sc_scatter_add(ms, dst, zeros_s, TM)
        mv, _ps_done = lax.optimization_barrier((mv, part_s))
        part_v = _sc_scatter_add(mv, dst, zeros_v, TV)

        if l < L - 1:
            s, vf, t = pl.pallas_call(
                _update_body,
                grid=(GN,),
                in_specs=[
                    pl.BlockSpec((2, BN, TM), lambda i: (0, i, 0)),
                    pl.BlockSpec((2, BN, TV), lambda i: (0, i, 0)),
                    pl.BlockSpec((BN, S), lambda i: (i, 0)),
                    pl.BlockSpec((BN, 12), lambda i: (i, 0)),
                    _full((S, S)),
                ],
                out_specs=[pl.BlockSpec((BN, S), lambda i: (i, 0)),
                           pl.BlockSpec((BN, 12), lambda i: (i, 0)),
                           pl.BlockSpec((BN, TG), lambda i: (i, 0))],
                out_shape=[jax.ShapeDtypeStruct((N, S), f32),
                           jax.ShapeDtypeStruct((N, 12), f32),
                           jax.ShapeDtypeStruct((N, TG), f32)],
            )(part_s, part_v, s, vf, wss[l + 1])
        else:
            out = pl.pallas_call(
                _final_body,
                grid=(GN,),
                in_specs=[
                    pl.BlockSpec((2, BN, TM), lambda i: (0, i, 0)),
                    pl.BlockSpec((2, BN, TV), lambda i: (0, i, 0)),
                    pl.BlockSpec((BN, S), lambda i: (i, 0)),
                    _full((S, 2 * LAT)),
                    _full((1, 2 * LAT)),
                    _full((2 * LAT, LAT)),
                    _full((1, LAT)),
                ],
                out_specs=pl.BlockSpec((BN, LAT), lambda i: (i, 0)),
                out_shape=jax.ShapeDtypeStruct((N, LAT), f32),
            )(part_s, part_v, s, al_w1, al_b1.reshape(1, 2 * LAT), al_w2,
              al_b2.reshape(1, LAT))
    return out
